# baseline (speedup 1.0000x reference)
import functools

import jax
import jax.numpy as jnp
from jax import lax
from jax.experimental import pallas as pl
from jax.experimental.pallas import tpu as pltpu
from jax.experimental.pallas import tpu_sc as plsc

_B = 32
_A = 3
_C5 = 85
_NCLS = 80
_HW = 4096
_IMGS = _B * _A
_NW = 32
_IPW = _IMGS // _NW
_BLK = 256
_NBLK = _HW // _BLK
_TBLK = _IPW * _NBLK
_AUX = 7

_LN2 = 0.6931471805599453
_SQRT2 = 1.4142135381698608


def _log_f32(x):
    bits = plsc.bitcast(x, jnp.int32)
    e = (bits >> 23) - 127
    mant = plsc.bitcast((bits & 0x007FFFFF) | 0x3F800000, jnp.float32)
    big = mant > _SQRT2
    mant = jnp.where(big, 0.5 * mant, mant)
    ef = (e + big.astype(jnp.int32)).astype(jnp.float32)
    u = mant - 1.0
    y = u / (u + 2.0)
    y2 = y * y
    poly = 1.0 + y2 * (1.0 / 3.0 + y2 * (0.2 + y2 * (1.0 / 7.0)))
    return ef * _LN2 + 2.0 * y * poly


def _prep_body(pred_ref, targ_ref, predl_ref, aux_ref):
    p = pred_ref[0].reshape(_C5, _HW)
    predl_ref[0] = p
    aux_ref[0, 5] = jnp.max(p[5:, :], axis=0)
    t = targ_ref[0, 0].reshape(_HW, _C5)
    for k in range(5):
        aux_ref[0, k] = t[:, k]
    tc = t[:, 5:]
    iot = lax.broadcasted_iota(jnp.int32, (_HW, _NCLS), 1)
    key = jnp.where(tc > 0.5, iot, 1000)
    kmin = jnp.min(key, axis=1)
    kmin = jnp.where(kmin > _NCLS - 1, 0, kmin)
    aux_ref[0, 6] = kmin.astype(jnp.float32)


def _prep_tc(pred, targ):
    return pl.pallas_call(
        _prep_body,
        grid=(_IMGS,),
        in_specs=[
            pl.BlockSpec((1, _C5, 64, 64), lambda i: (i // _A, i % _A, 0, 0)),
            pl.BlockSpec((1, 1, 64, 64, _C5),
                         lambda i: (i // _A, i % _A, 0, 0, 0)),
        ],
        out_specs=[
            pl.BlockSpec((1, _C5, _HW), lambda i: (i, 0, 0)),
            pl.BlockSpec((1, _AUX, _HW), lambda i: (i, 0, 0)),
        ],
        out_shape=[
            jax.ShapeDtypeStruct((_IMGS, _C5, _HW), jnp.float32),
            jax.ShapeDtypeStruct((_IMGS, _AUX, _HW), jnp.float32),
        ],
    )(pred, targ)


def _yolo_body(pred_hbm, aux_hbm, out_hbm, p0, p1, t0, t1, obuf,
               sp0, sp1, st0, st1):
    cid = lax.axis_index("c")
    sid = lax.axis_index("s")
    wid = sid * 2 + cid
    iota16 = lax.iota(jnp.int32, 16)

    def dmas(t, pbuf, tbuf, sp, st):
        img = wid * _IPW + t // _NBLK
        n0 = (t % _NBLK) * _BLK
        cp = pltpu.make_async_copy(
            pred_hbm.at[img, :, pl.ds(n0, _BLK)], pbuf, sp)
        ct = pltpu.make_async_copy(
            aux_hbm.at[img, :, pl.ds(n0, _BLK)], tbuf, st)
        return cp, ct

    def issue(t, pbuf, tbuf, sp, st):
        cp, ct = dmas(t, pbuf, tbuf, sp, st)
        cp.start()
        ct.start()

    def wait(t, pbuf, tbuf, sp, st):
        cp, ct = dmas(t, pbuf, tbuf, sp, st)
        cp.wait()
        ct.wait()

    def compute_block(pbuf, tbuf, carry):
        def group(gi, carry):
            acc_loc, acc_conf, acc_cls = carry
            base = gi * 16
            rows = base + iota16
            sb = [pbuf[k, pl.ds(base, 16)] for k in range(5)]
            tb = [tbuf[k, pl.ds(base, 16)] for k in range(5)]
            obj = tb[4]
            sig0 = 1.0 / (1.0 + jnp.exp(-sb[0]))
            sig1 = 1.0 / (1.0 + jnp.exp(-sb[1]))
            d0 = sig0 - tb[0]
            d1 = sig1 - tb[1]
            d2 = sb[2] - tb[2]
            d3 = sb[3] - tb[3]
            acc_loc = acc_loc + obj * (d0 * d0 + d1 * d1 + d2 * d2 + d3 * d3)
            z = sb[4]
            az = jnp.abs(z)
            la = 0.5 * (z + az) + _log_f32(1.0 + jnp.exp(-az))
            acc_conf = acc_conf + (0.5 + 0.5 * obj) * la - obj * z
            m = tbuf[5, pl.ds(base, 16)]
            kidx = tbuf[6, pl.ds(base, 16)].astype(jnp.int32)
            ss = [jnp.zeros((16,), jnp.float32) for _ in range(4)]
            for k in range(_NCLS):
                j = k & 3
                s = pbuf[5 + k, pl.ds(base, 16)]
                ss[j] = ss[j] + jnp.exp(s - m)
            ssum = (ss[0] + ss[1]) + (ss[2] + ss[3])
            pick = plsc.load_gather(pbuf, [5 + kidx, rows])
            lse = m + _log_f32(ssum)
            acc_cls = acc_cls + obj * (lse - pick)
            return acc_loc, acc_conf, acc_cls

        return lax.fori_loop(0, _BLK // 16, group, carry)

    issue(0, p0, t0, sp0, st0)

    def pair(tp, carry):
        e = 2 * tp
        wait(e, p0, t0, sp0, st0)
        issue(e + 1, p1, t1, sp1, st1)
        carry = compute_block(p0, t0, carry)
        o = e + 1
        wait(o, p1, t1, sp1, st1)

        @pl.when(o + 1 < _TBLK)
        def _():
            issue(o + 1, p0, t0, sp0, st0)

        carry = compute_block(p1, t1, carry)
        return carry

    zero = jnp.zeros((16,), jnp.float32)
    acc_loc, acc_conf, acc_cls = lax.fori_loop(
        0, _TBLK // 2, pair, (zero, zero, zero))
    obuf[...] = 5.0 * acc_loc + acc_conf + acc_cls
    pltpu.sync_copy(obuf, out_hbm.at[wid])


@functools.cache
def _yolo_sc():
    return pl.kernel(
        _yolo_body,
        out_type=jax.ShapeDtypeStruct((_NW, 16), jnp.float32),
        mesh=plsc.VectorSubcoreMesh(core_axis_name="c", subcore_axis_name="s"),
        compiler_params=pltpu.CompilerParams(needs_layout_passes=False),
        scratch_types=[
            pltpu.VMEM((_C5, _BLK), jnp.float32),
            pltpu.VMEM((_C5, _BLK), jnp.float32),
            pltpu.VMEM((_AUX, _BLK), jnp.float32),
            pltpu.VMEM((_AUX, _BLK), jnp.float32),
            pltpu.VMEM((16,), jnp.float32),
            pltpu.SemaphoreType.DMA,
            pltpu.SemaphoreType.DMA,
            pltpu.SemaphoreType.DMA,
            pltpu.SemaphoreType.DMA,
        ],
    )


@jax.jit
def kernel(predictions, targets):
    predl, aux = _prep_tc(predictions, targets)
    partials = _yolo_sc()(predl, aux)
    return jnp.sum(partials) / _B


# speedup vs baseline: 1.9822x; 1.9822x over previous
import functools

import jax
import jax.numpy as jnp
from jax import lax
from jax.experimental import pallas as pl
from jax.experimental.pallas import tpu as pltpu
from jax.experimental.pallas import tpu_sc as plsc

_B = 32
_A = 3
_C5 = 85
_NCLS = 80
_HW = 4096
_IMGS = _B * _A
_NW = 32
_IPW = _IMGS // _NW
_BLK = 256
_NBLK = _HW // _BLK
_TBLK = _IPW * _NBLK
_AUX = 10

_LN2 = 0.6931471805599453
_SQRT2 = 1.4142135381698608


def _log_f32(x):
    bits = plsc.bitcast(x, jnp.int32)
    e = (bits >> 23) - 127
    mant = plsc.bitcast((bits & 0x007FFFFF) | 0x3F800000, jnp.float32)
    big = mant > _SQRT2
    mant = jnp.where(big, 0.5 * mant, mant)
    ef = (e + big.astype(jnp.int32)).astype(jnp.float32)
    u = mant - 1.0
    y = u / (u + 2.0)
    y2 = y * y
    poly = 1.0 + y2 * (1.0 / 3.0 + y2 * (0.2 + y2 * (1.0 / 7.0)))
    return ef * _LN2 + 2.0 * y * poly


def _prep_body(pred_ref, predl_ref, maxp_ref):
    p = pred_ref[0].reshape(_C5, _HW)
    predl_ref[0] = p
    maxp_ref[0, 0] = jnp.max(p[5:, :], axis=0)


def _prep_tc(pred):
    return pl.pallas_call(
        _prep_body,
        grid=(_IMGS,),
        in_specs=[
            pl.BlockSpec((1, _C5, 64, 64), lambda i: (i, 0, 0, 0)),
        ],
        out_specs=[
            pl.BlockSpec((1, _C5, _HW), lambda i: (i, 0, 0)),
            pl.BlockSpec((1, 1, _HW), lambda i: (i, 0, 0)),
        ],
        out_shape=[
            jax.ShapeDtypeStruct((_IMGS, _C5, _HW), jnp.float32),
            jax.ShapeDtypeStruct((_IMGS, 1, _HW), jnp.float32),
        ],
    )(pred)


def _yolo_body(pred_hbm, targ_hbm, maxp_hbm, out_hbm, p0, p1, t0, t1,
               m0, m1, obuf, sp0, sp1, st0, st1, sm0, sm1):
    cid = lax.axis_index("c")
    sid = lax.axis_index("s")
    wid = sid * 2 + cid
    iota16 = lax.iota(jnp.int32, 16)

    def dmas(t, pbuf, tbuf, mbuf, sp, st, sm):
        img = wid * _IPW + t // _NBLK
        n0 = (t % _NBLK) * _BLK
        cp = pltpu.make_async_copy(
            pred_hbm.at[img, :, pl.ds(n0, _BLK)], pbuf, sp)
        ct = pltpu.make_async_copy(
            targ_hbm.at[img, :, pl.ds(n0, _BLK)], tbuf, st)
        cm = pltpu.make_async_copy(
            maxp_hbm.at[img, 0, pl.ds(n0, _BLK)], mbuf, sm)
        return cp, ct, cm

    def issue(t, pbuf, tbuf, mbuf, sp, st, sm):
        for c in dmas(t, pbuf, tbuf, mbuf, sp, st, sm):
            c.start()

    def wait(t, pbuf, tbuf, mbuf, sp, st, sm):
        for c in dmas(t, pbuf, tbuf, mbuf, sp, st, sm):
            c.wait()

    def compute_block(pbuf, tbuf, mbuf, carry):
        def group(gi, carry):
            acc_loc, acc_conf, acc_cls = carry
            base = gi * 16
            rows = base + iota16
            sb = [pbuf[k, pl.ds(base, 16)] for k in range(5)]
            tb = [tbuf[k, pl.ds(base, 16)] for k in range(5)]
            obj = tb[4]
            sig0 = 1.0 / (1.0 + jnp.exp(-sb[0]))
            sig1 = 1.0 / (1.0 + jnp.exp(-sb[1]))
            d0 = sig0 - tb[0]
            d1 = sig1 - tb[1]
            d2 = sb[2] - tb[2]
            d3 = sb[3] - tb[3]
            acc_loc = acc_loc + obj * (d0 * d0 + d1 * d1 + d2 * d2 + d3 * d3)
            z = sb[4]
            az = jnp.abs(z)
            la = 0.5 * (z + az) + _log_f32(1.0 + jnp.exp(-az))
            acc_conf = acc_conf + (0.5 + 0.5 * obj) * la - obj * z
            m = mbuf[pl.ds(base, 16)]
            ss = [jnp.zeros((16,), jnp.float32) for _ in range(4)]
            km = [jnp.full((16,), 1000.0, jnp.float32) for _ in range(4)]
            for k in range(_NCLS):
                j = k & 3
                s = pbuf[5 + k, pl.ds(base, 16)]
                tt = tbuf[5 + k, pl.ds(base, 16)]
                ss[j] = ss[j] + jnp.exp(s - m)
                key = (1.0 - tt) * 1000.0 + k
                km[j] = jnp.minimum(km[j], key)
            ssum = (ss[0] + ss[1]) + (ss[2] + ss[3])
            kmin = jnp.minimum(jnp.minimum(km[0], km[1]),
                               jnp.minimum(km[2], km[3]))
            kidx = kmin.astype(jnp.int32)
            kidx = jnp.where(kidx > _NCLS - 1, 0, kidx)
            pick = plsc.load_gather(pbuf, [5 + kidx, rows])
            lse = m + _log_f32(ssum)
            acc_cls = acc_cls + obj * (lse - pick)
            return acc_loc, acc_conf, acc_cls

        return lax.fori_loop(0, _BLK // 16, group, carry)

    issue(0, p0, t0, m0, sp0, st0, sm0)

    def pair(tp, carry):
        e = 2 * tp
        wait(e, p0, t0, m0, sp0, st0, sm0)
        issue(e + 1, p1, t1, m1, sp1, st1, sm1)
        carry = compute_block(p0, t0, m0, carry)
        o = e + 1
        wait(o, p1, t1, m1, sp1, st1, sm1)

        @pl.when(o + 1 < _TBLK)
        def _():
            issue(o + 1, p0, t0, m0, sp0, st0, sm0)

        carry = compute_block(p1, t1, m1, carry)
        return carry

    zero = jnp.zeros((16,), jnp.float32)
    acc_loc, acc_conf, acc_cls = lax.fori_loop(
        0, _TBLK // 2, pair, (zero, zero, zero))
    obuf[...] = 5.0 * acc_loc + acc_conf + acc_cls
    pltpu.sync_copy(obuf, out_hbm.at[wid])


@functools.cache
def _yolo_sc():
    return pl.kernel(
        _yolo_body,
        out_type=jax.ShapeDtypeStruct((_NW, 16), jnp.float32),
        mesh=plsc.VectorSubcoreMesh(core_axis_name="c", subcore_axis_name="s"),
        compiler_params=pltpu.CompilerParams(needs_layout_passes=False),
        scratch_types=[
            pltpu.VMEM((_C5, _BLK), jnp.float32),
            pltpu.VMEM((_C5, _BLK), jnp.float32),
            pltpu.VMEM((_C5, _BLK), jnp.float32),
            pltpu.VMEM((_C5, _BLK), jnp.float32),
            pltpu.VMEM((_BLK,), jnp.float32),
            pltpu.VMEM((_BLK,), jnp.float32),
            pltpu.VMEM((16,), jnp.float32),
            pltpu.SemaphoreType.DMA,
            pltpu.SemaphoreType.DMA,
            pltpu.SemaphoreType.DMA,
            pltpu.SemaphoreType.DMA,
            pltpu.SemaphoreType.DMA,
            pltpu.SemaphoreType.DMA,
        ],
    )


@jax.jit
def kernel(predictions, targets):
    pred4 = predictions.reshape(_IMGS, _C5, 64, 64)
    targt = jnp.transpose(targets.reshape(_IMGS, _HW, _C5), (0, 2, 1))
    predl, maxp = _prep_tc(pred4)
    partials = _yolo_sc()(predl, targt, maxp)
    return jnp.sum(partials) / _B


# speedup vs baseline: 1.9836x; 1.0007x over previous
import functools

import jax
import jax.numpy as jnp
from jax import lax
from jax.experimental import pallas as pl
from jax.experimental.pallas import tpu as pltpu
from jax.experimental.pallas import tpu_sc as plsc

_B = 32
_A = 3
_C5 = 85
_NCLS = 80
_HW = 4096
_IMGS = _B * _A
_NW = 32
_IPW = _IMGS // _NW
_BLK = 256
_NBLK = _HW // _BLK
_TBLK = _IPW * _NBLK

_LN2 = 0.6931471805599453
_SQRT2 = 1.4142135381698608


def _log_f32(x):
    bits = plsc.bitcast(x, jnp.int32)
    e = (bits >> 23) - 127
    mant = plsc.bitcast((bits & 0x007FFFFF) | 0x3F800000, jnp.float32)
    big = mant > _SQRT2
    mant = jnp.where(big, 0.5 * mant, mant)
    ef = (e + big.astype(jnp.int32)).astype(jnp.float32)
    u = mant - 1.0
    y = u / (u + 2.0)
    y2 = y * y
    poly = 1.0 + y2 * (1.0 / 3.0 + y2 * (0.2 + y2 * (1.0 / 7.0)))
    return ef * _LN2 + 2.0 * y * poly


def _prep_body(pred_ref, predl_ref, maxp_ref):
    p = pred_ref[0].reshape(_C5, _HW)
    predl_ref[0] = p
    maxp_ref[0, 0] = jnp.max(p[5:, :], axis=0)


def _prep_tc(pred):
    return pl.pallas_call(
        _prep_body,
        grid=(_IMGS,),
        in_specs=[
            pl.BlockSpec((1, _C5, 64, 64), lambda i: (i, 0, 0, 0)),
        ],
        out_specs=[
            pl.BlockSpec((1, _C5, _HW), lambda i: (i, 0, 0)),
            pl.BlockSpec((1, 1, _HW), lambda i: (i, 0, 0)),
        ],
        out_shape=[
            jax.ShapeDtypeStruct((_IMGS, _C5, _HW), jnp.float32),
            jax.ShapeDtypeStruct((_IMGS, 1, _HW), jnp.float32),
        ],
    )(pred)


def _yolo_body(pred_hbm, targ_hbm, maxp_hbm, out_hbm, p0, p1, t0, t1,
               m0, m1, obuf, sp0, sp1, st0, st1, sm0, sm1):
    cid = lax.axis_index("c")
    sid = lax.axis_index("s")
    wid = sid * 2 + cid
    iota16 = lax.iota(jnp.int32, 16)

    def dmas(t, pbuf, tbuf, mbuf, sp, st, sm):
        img = wid * _IPW + t // _NBLK
        n0 = (t % _NBLK) * _BLK
        cp = pltpu.make_async_copy(
            pred_hbm.at[img, :, pl.ds(n0, _BLK)], pbuf, sp)
        ct = pltpu.make_async_copy(
            targ_hbm.at[img, :, pl.ds(n0, _BLK)], tbuf, st)
        cm = pltpu.make_async_copy(
            maxp_hbm.at[img, 0, pl.ds(n0, _BLK)], mbuf, sm)
        return cp, ct, cm

    def issue(t, pbuf, tbuf, mbuf, sp, st, sm):
        for c in dmas(t, pbuf, tbuf, mbuf, sp, st, sm):
            c.start()

    def wait(t, pbuf, tbuf, mbuf, sp, st, sm):
        for c in dmas(t, pbuf, tbuf, mbuf, sp, st, sm):
            c.wait()

    def compute_block(pbuf, tbuf, mbuf, carry):
        def group(gi, carry):
            acc_loc, acc_conf, acc_cls = carry
            base = gi * 16
            rows = base + iota16
            sb = [pbuf[k, pl.ds(base, 16)] for k in range(5)]
            tb = [tbuf[k, pl.ds(base, 16)] for k in range(5)]
            obj = tb[4]
            sig0 = 1.0 / (1.0 + jnp.exp(-sb[0]))
            sig1 = 1.0 / (1.0 + jnp.exp(-sb[1]))
            d0 = sig0 - tb[0]
            d1 = sig1 - tb[1]
            d2 = sb[2] - tb[2]
            d3 = sb[3] - tb[3]
            acc_loc = acc_loc + obj * (d0 * d0 + d1 * d1 + d2 * d2 + d3 * d3)
            z = sb[4]
            az = jnp.abs(z)
            la = 0.5 * (z + az) + _log_f32(1.0 + jnp.exp(-az))
            acc_conf = acc_conf + (0.5 + 0.5 * obj) * la - obj * z
            m = mbuf[pl.ds(base, 16)]
            ss = [jnp.zeros((16,), jnp.float32) for _ in range(4)]
            km = [jnp.full((16,), 1000.0, jnp.float32) for _ in range(4)]
            for k in range(_NCLS):
                j = k & 3
                s = pbuf[5 + k, pl.ds(base, 16)]
                tt = tbuf[5 + k, pl.ds(base, 16)]
                ss[j] = ss[j] + jnp.exp(s - m)
                key = (1.0 - tt) * 1000.0 + k
                km[j] = jnp.minimum(km[j], key)
            ssum = (ss[0] + ss[1]) + (ss[2] + ss[3])
            kmin = jnp.minimum(jnp.minimum(km[0], km[1]),
                               jnp.minimum(km[2], km[3]))
            kidx = kmin.astype(jnp.int32)
            kidx = jnp.where(kidx > _NCLS - 1, 0, kidx)
            pick = plsc.load_gather(pbuf, [5 + kidx, rows])
            lse = m + _log_f32(ssum)
            acc_cls = acc_cls + obj * (lse - pick)
            return acc_loc, acc_conf, acc_cls

        return lax.fori_loop(0, _BLK // 16, group, carry)

    issue(0, p0, t0, m0, sp0, st0, sm0)

    def pair(tp, carry):
        e = 2 * tp
        wait(e, p0, t0, m0, sp0, st0, sm0)
        issue(e + 1, p1, t1, m1, sp1, st1, sm1)
        carry = compute_block(p0, t0, m0, carry)
        o = e + 1
        wait(o, p1, t1, m1, sp1, st1, sm1)

        @pl.when(o + 1 < _TBLK)
        def _():
            issue(o + 1, p0, t0, m0, sp0, st0, sm0)

        carry = compute_block(p1, t1, m1, carry)
        return carry

    zero = jnp.zeros((16,), jnp.float32)
    acc_loc, acc_conf, acc_cls = lax.fori_loop(
        0, _TBLK // 2, pair, (zero, zero, zero))
    obuf[...] = 5.0 * acc_loc + acc_conf + acc_cls
    pltpu.sync_copy(obuf, out_hbm.at[wid])


@functools.cache
def _yolo_sc():
    return pl.kernel(
        _yolo_body,
        out_type=jax.ShapeDtypeStruct((_NW, 16), jnp.float32),
        mesh=plsc.VectorSubcoreMesh(core_axis_name="c", subcore_axis_name="s"),
        compiler_params=pltpu.CompilerParams(needs_layout_passes=False),
        scratch_types=[
            pltpu.VMEM((_C5, _BLK), jnp.float32),
            pltpu.VMEM((_C5, _BLK), jnp.float32),
            pltpu.VMEM((_C5, _BLK), jnp.float32),
            pltpu.VMEM((_C5, _BLK), jnp.float32),
            pltpu.VMEM((_BLK,), jnp.float32),
            pltpu.VMEM((_BLK,), jnp.float32),
            pltpu.VMEM((16,), jnp.float32),
            pltpu.SemaphoreType.DMA,
            pltpu.SemaphoreType.DMA,
            pltpu.SemaphoreType.DMA,
            pltpu.SemaphoreType.DMA,
            pltpu.SemaphoreType.DMA,
            pltpu.SemaphoreType.DMA,
        ],
    )


@jax.jit
def kernel(predictions, targets):
    pred4 = predictions.reshape(_IMGS, _C5, 64, 64)
    targt = jnp.transpose(targets.reshape(_IMGS, _HW, _C5), (0, 2, 1))
    predl, maxp = _prep_tc(pred4)
    partials = _yolo_sc()(predl, targt, maxp)
    return jnp.sum(partials) / _B
